# baseline (speedup 1.0000x reference)
import functools

import jax
import jax.numpy as jnp
from jax.experimental import pallas as pl
from jax.experimental.pallas import tpu as pltpu


def _ffn_kernel(x_ref, w1_ref, b1_ref, w2_ref, b2_ref, out_ref):
    j = pl.program_id(1)
    h = jnp.dot(x_ref[...], w1_ref[...], preferred_element_type=jnp.float32)
    h = jnp.maximum(h + b1_ref[...], 0.0)
    partial = jnp.dot(h, w2_ref[...], preferred_element_type=jnp.float32)

    @pl.when(j == 0)
    def _():
        out_ref[...] = partial + b2_ref[...]

    @pl.when(j != 0)
    def _():
        out_ref[...] += partial


@functools.partial(jax.jit, static_argnames=("blk_m", "blk_ff"))
def _ffn(x, W1, b1, W2, b2, blk_m=512, blk_ff=1024):
    n_tok, d_model = x.shape
    d_ff = W1.shape[1]
    blk_m = min(blk_m, n_tok)
    blk_ff = min(blk_ff, d_ff)
    grid = (n_tok // blk_m, d_ff // blk_ff)
    return pl.pallas_call(
        _ffn_kernel,
        grid=grid,
        in_specs=[
            pl.BlockSpec((blk_m, d_model), lambda i, j: (i, 0)),
            pl.BlockSpec((d_model, blk_ff), lambda i, j: (0, j)),
            pl.BlockSpec((blk_ff,), lambda i, j: (j,)),
            pl.BlockSpec((blk_ff, d_model), lambda i, j: (j, 0)),
            pl.BlockSpec((d_model,), lambda i, j: (0,)),
        ],
        out_specs=pl.BlockSpec((blk_m, d_model), lambda i, j: (i, 0)),
        out_shape=jax.ShapeDtypeStruct((n_tok, d_model), jnp.float32),
        compiler_params=pltpu.CompilerParams(
            dimension_semantics=("parallel", "arbitrary"),
        ),
    )(x, W1, b1, W2, b2)


def kernel(x, W1, b1, W2, b2):
    return _ffn(x, W1, b1, W2, b2)


# speedup vs baseline: 1.0405x; 1.0405x over previous
import functools

import jax
import jax.numpy as jnp
from jax.experimental import pallas as pl
from jax.experimental.pallas import tpu as pltpu


def _ffn_kernel(x_ref, w1_ref, b1_ref, w2_ref, b2_ref, out_ref):
    j = pl.program_id(1)
    h = jnp.dot(x_ref[...], w1_ref[...], preferred_element_type=jnp.float32)
    h = jnp.maximum(h + b1_ref[...], 0.0)
    partial = jnp.dot(h, w2_ref[...], preferred_element_type=jnp.float32)

    @pl.when(j == 0)
    def _():
        out_ref[...] = partial + b2_ref[...]

    @pl.when(j != 0)
    def _():
        out_ref[...] += partial


@functools.partial(jax.jit, static_argnames=("blk_m", "blk_ff"))
def _ffn(x, W1, b1, W2, b2, blk_m=1024, blk_ff=512):
    n_tok, d_model = x.shape
    d_ff = W1.shape[1]
    blk_m = min(blk_m, n_tok)
    blk_ff = min(blk_ff, d_ff)
    grid = (n_tok // blk_m, d_ff // blk_ff)
    return pl.pallas_call(
        _ffn_kernel,
        grid=grid,
        in_specs=[
            pl.BlockSpec((blk_m, d_model), lambda i, j: (i, 0)),
            pl.BlockSpec((d_model, blk_ff), lambda i, j: (0, j)),
            pl.BlockSpec((blk_ff,), lambda i, j: (j,)),
            pl.BlockSpec((blk_ff, d_model), lambda i, j: (j, 0)),
            pl.BlockSpec((d_model,), lambda i, j: (0,)),
        ],
        out_specs=pl.BlockSpec((blk_m, d_model), lambda i, j: (i, 0)),
        out_shape=jax.ShapeDtypeStruct((n_tok, d_model), jnp.float32),
        compiler_params=pltpu.CompilerParams(
            dimension_semantics=("parallel", "arbitrary"),
        ),
    )(x, W1, b1, W2, b2)


def kernel(x, W1, b1, W2, b2):
    return _ffn(x, W1, b1, W2, b2)
